# Initial kernel scaffold; baseline (speedup 1.0000x reference)
import jax
import jax.numpy as jnp
from jax.experimental import pallas as pl


def kernel(x, table, W1, b1, W2, b2):
    raise NotImplementedError("write your pallas kernel here")


# speedup vs baseline: 1.7531x; 1.7531x over previous
import functools

import jax
import jax.numpy as jnp
from jax import lax
from jax.experimental import pallas as pl
from jax.experimental.pallas import tpu as pltpu
from jax.experimental.pallas import tpu_sc as plsc

EMB = 32
HIDDEN = 128
CLASS = 100
BATCH = 4096
SEQ = 200

NW = 32
BPW = BATCH // NW
LANES = 16
HALF = EMB // LANES


def _pool_body(xw_hbm, table_hbm, out_hbm, idx_v, buf0, buf1, acc_v, sem0, sem1):
    w = lax.axis_index("c") * 16 + lax.axis_index("s")
    pltpu.sync_copy(xw_hbm.at[w], idx_v)

    zero = jnp.zeros((LANES,), jnp.float32)
    for i in range(BPW):
        for h in range(HALF):
            acc_v[i, h * LANES:(h + 1) * LANES] = zero

    def start_gather(j, buf, sem):
        pltpu.make_async_copy(table_hbm.at[idx_v.at[j]], buf, sem).start()

    def wait_gather(j, buf, sem):
        pltpu.make_async_copy(table_hbm.at[idx_v.at[j]], buf, sem).wait()

    def accumulate(buf):
        for i in range(BPW):
            for h in range(HALF):
                plsc.addupdate(
                    acc_v.at[i, pl.ds(h * LANES, LANES)],
                    buf[i, h * LANES:(h + 1) * LANES],
                )

    start_gather(0, buf0, sem0)
    start_gather(1, buf1, sem1)

    def body(t, carry):
        j0 = 2 * t
        wait_gather(j0, buf0, sem0)
        accumulate(buf0)
        start_gather(j0 + 2, buf0, sem0)
        wait_gather(j0 + 1, buf1, sem1)
        accumulate(buf1)
        start_gather(j0 + 3, buf1, sem1)
        return carry

    lax.fori_loop(0, SEQ // 2 - 1, body, 0)

    wait_gather(SEQ - 2, buf0, sem0)
    accumulate(buf0)
    wait_gather(SEQ - 1, buf1, sem1)
    accumulate(buf1)

    pltpu.sync_copy(acc_v, out_hbm.at[pl.ds(w * BPW, BPW), :])


_pool_call = functools.partial(
    pl.kernel,
    mesh=plsc.VectorSubcoreMesh(core_axis_name="c", subcore_axis_name="s"),
    out_type=jax.ShapeDtypeStruct((BATCH, EMB), jnp.float32),
    scratch_types=[
        pltpu.VMEM((SEQ, BPW), jnp.int32),
        pltpu.VMEM((BPW, EMB), jnp.float32),
        pltpu.VMEM((BPW, EMB), jnp.float32),
        pltpu.VMEM((BPW, EMB), jnp.float32),
        pltpu.SemaphoreType.DMA,
        pltpu.SemaphoreType.DMA,
    ],
    compiler_params=pltpu.CompilerParams(use_tc_tiling_on_sc=False),
)(_pool_body)


def _mlp_body(p_ref, w1_ref, b1_ref, w2_ref, b2_ref, o_ref):
    h = jnp.dot(p_ref[:], w1_ref[:], preferred_element_type=jnp.float32)
    h = jnp.maximum(h + b1_ref[:], 0.0)
    o_ref[:] = jnp.dot(h, w2_ref[:], preferred_element_type=jnp.float32) + b2_ref[:]


def _mlp_call(pooled, w1, b1, w2, b2):
    return pl.pallas_call(
        _mlp_body,
        out_shape=jax.ShapeDtypeStruct((BATCH, HIDDEN), jnp.float32),
    )(pooled, w1, b1, w2, b2)


@jax.jit
def kernel(x, table, W1, b1, W2, b2):
    xw = x.astype(jnp.int32).T.reshape(SEQ, NW, BPW).transpose(1, 0, 2)
    pooled_sum = _pool_call(xw, table)
    w1s = W1 * (1.0 / SEQ)
    b1r = b1.reshape(1, HIDDEN)
    w2p = jnp.pad(W2, ((0, 0), (0, HIDDEN - CLASS)))
    b2p = jnp.pad(b2, (0, HIDDEN - CLASS)).reshape(1, HIDDEN)
    out = _mlp_call(pooled_sum, w1s, b1r, w2p, b2p)
    return out[:, :CLASS]
